# baseline (speedup 1.0000x reference)
import functools

import jax
import jax.numpy as jnp
from jax import lax
from jax.experimental import pallas as pl
from jax.experimental.pallas import tpu as pltpu
from jax.experimental.pallas import tpu_sc as plsc

N_NODES = 10000
DEG = 32
E = N_NODES * DEG
D_FEAT = 128
D_COORD = 4
UNITS = 128

NC = 2
NS = 16
NW = NC * NS
CHUNK = 128
NCHUNK_TOTAL = E // CHUNK
N_PAD = 10240
ROWS_PER_TILE = N_PAD // NS
IDX_WIN = 80
NCHUNK_PAD = 2576

BN = 400
BE = BN * DEG

PHASE_NODES = (8400, 1600)


def _feats_body(x_ref, wp_ref, bp_ref, coord_ref, sel_ref, out_ref):
    x = x_ref[...]
    hp = jnp.dot(x, wp_ref[...], preferred_element_type=jnp.float32)
    hp = hp + bp_ref[...]
    cdall = jax.lax.dot_general(
        coord_ref[...], sel_ref[...], (((0,), (0,)), ((), ())),
        preferred_element_type=jnp.float32)
    acc = jnp.zeros((BE, UNITS), jnp.float32)
    for d in range(D_COORD):
        hd = hp[:, d * UNITS:(d + 1) * UNITS]
        hrep = jnp.broadcast_to(hd[:, None, :], (BN, DEG, UNITS))
        hrep = hrep.reshape(BE, UNITS)
        acc = acc + hrep * cdall[:, d * UNITS:(d + 1) * UNITS]
    out_ref[...] = acc


def _compute_feats(node_features, coordT, Wp, bp, sel, n_blocks, blk_off):
    return pl.pallas_call(
        _feats_body,
        grid=(n_blocks,),
        in_specs=[
            pl.BlockSpec((BN, D_FEAT), lambda i: (i + blk_off, 0)),
            pl.BlockSpec((D_FEAT, D_COORD * UNITS), lambda i: (0, 0)),
            pl.BlockSpec((1, D_COORD * UNITS), lambda i: (0, 0)),
            pl.BlockSpec((D_COORD, BE), lambda i: (0, i + blk_off)),
            pl.BlockSpec((D_COORD, D_COORD * UNITS), lambda i: (0, 0)),
        ],
        out_specs=pl.BlockSpec((BE, UNITS), lambda i: (i, 0)),
        out_shape=jax.ShapeDtypeStruct((n_blocks * BE, UNITS), jnp.float32),
    )(node_features, Wp, bp, coordT, sel)


def _sc_scatter_body(chunk0, nchunk, nextra,
                     feats_hbm, idx_hbm, out_hbm,
                     idx_v, buf_v, acc_sh, sem0, sem1):
    cid = lax.axis_index("c")
    sid = lax.axis_index("s")
    wid = cid * NS + sid

    zv = jnp.zeros((16,), jnp.float32)

    @pl.loop(0, 128)
    def _zero_rows(r):
        for s in range(2):
            for c in range(UNITS // 16):
                buf_v[s, r, pl.ds(c * 16, 16)] = zv

    row0 = pl.multiple_of(sid * ROWS_PER_TILE, 8)
    for k in range(ROWS_PER_TILE // 128):
        pltpu.sync_copy(buf_v.at[k % 2],
                        acc_sh.at[pl.ds(row0 + k * 128, 128)])

    base = nchunk * wid + jnp.minimum(wid, nextra)

    gbase = base + chunk0
    base_al = pl.multiple_of((gbase // 8) * 8, 8)
    dlt = gbase - base_al
    pltpu.sync_copy(idx_hbm.at[pl.ds(base_al, IDX_WIN)], idx_v)

    plsc.subcore_barrier()

    neven = nchunk & ~1
    b0 = buf_v.at[0]
    b1 = buf_v.at[1]
    pltpu.async_copy(feats_hbm.at[base], b0, sem0)
    pltpu.async_copy(feats_hbm.at[base + 1], b1, sem1)

    @pl.loop(0, neven // 2 - 1)
    def _main(j):
        c = j * 2
        pltpu.make_async_copy(feats_hbm.at[base + c], b0, sem0).wait()
        pltpu.sync_copy(b0, acc_sh.at[idx_v.at[dlt + c]], add=True)
        pltpu.async_copy(feats_hbm.at[base + c + 2], b0, sem0)
        pltpu.make_async_copy(feats_hbm.at[base + c + 1], b1, sem1).wait()
        pltpu.sync_copy(b1, acc_sh.at[idx_v.at[dlt + c + 1]], add=True)
        pltpu.async_copy(feats_hbm.at[base + c + 3], b1, sem1)

    last = neven - 2
    pltpu.make_async_copy(feats_hbm.at[base + last], b0, sem0).wait()
    pltpu.sync_copy(b0, acc_sh.at[idx_v.at[dlt + last]], add=True)
    if nchunk & 1:
        pltpu.async_copy(feats_hbm.at[base + neven], b0, sem0)
    pltpu.make_async_copy(feats_hbm.at[base + last + 1], b1, sem1).wait()
    pltpu.sync_copy(b1, acc_sh.at[idx_v.at[dlt + last + 1]], add=True)
    if nchunk & 1:
        pltpu.make_async_copy(feats_hbm.at[base + neven], b0, sem0).wait()
        pltpu.sync_copy(b0, acc_sh.at[idx_v.at[dlt + neven]], add=True)

    if nextra:
        @pl.when(wid < nextra)
        def _extra():
            pltpu.sync_copy(feats_hbm.at[base + nchunk], b0)
            pltpu.sync_copy(b0, acc_sh.at[idx_v.at[dlt + nchunk]], add=True)

    plsc.subcore_barrier()

    pltpu.sync_copy(acc_sh.at[pl.ds(row0, ROWS_PER_TILE)],
                    out_hbm.at[cid, pl.ds(row0, ROWS_PER_TILE)])


@functools.lru_cache(maxsize=4)
def _sc_scatter(chunk0, nchunks_phase):
    nchunk = nchunks_phase // NW
    nextra = nchunks_phase - nchunk * NW
    return pl.kernel(
        functools.partial(_sc_scatter_body, chunk0, nchunk, nextra),
        out_type=jax.ShapeDtypeStruct((NC, N_PAD, UNITS), jnp.float32),
        mesh=plsc.VectorSubcoreMesh(
            core_axis_name="c", subcore_axis_name="s",
            num_cores=NC, num_subcores=NS),
        scratch_types=[
            pltpu.VMEM((IDX_WIN, CHUNK), jnp.int32),
            pltpu.VMEM((2, CHUNK, UNITS), jnp.float32),
            pltpu.VMEM_SHARED((N_PAD, UNITS), jnp.float32),
            pltpu.SemaphoreType.DMA,
            pltpu.SemaphoreType.DMA,
        ],
    )


def _combine_body(*refs):
    o_ref = refs[-1]
    acc = None
    for p_ref in refs[:-1]:
        t = p_ref[0] + p_ref[1]
        acc = t if acc is None else acc + t
    o_ref[...] = acc


def _combine(partials):
    nb = 10
    rb = N_NODES // nb
    return pl.pallas_call(
        _combine_body,
        grid=(nb,),
        in_specs=[pl.BlockSpec((NC, rb, UNITS), lambda i: (0, i, 0))
                  for _ in partials],
        out_specs=pl.BlockSpec((rb, UNITS), lambda i: (i, 0)),
        out_shape=jax.ShapeDtypeStruct((N_NODES, UNITS), jnp.float32),
    )(*partials)


def kernel(node_features, coord_features, indices, row_splits, W, b):
    del row_splits
    Wp = W.reshape(D_FEAT, UNITS, D_COORD).transpose(0, 2, 1)
    Wp = Wp.reshape(D_FEAT, D_COORD * UNITS)
    bp = b.reshape(UNITS, D_COORD).T.reshape(1, D_COORD * UNITS)
    sel = jnp.repeat(jnp.eye(D_COORD, dtype=jnp.float32), UNITS, axis=1)
    coordT = coord_features.T

    idx2 = jnp.concatenate(
        [indices, jnp.zeros((NCHUNK_PAD * CHUNK - E,), jnp.int32)]
    ).reshape(NCHUNK_PAD, CHUNK)

    feats3s, scs = [], []
    blk_off = 0
    chunk0 = 0
    for n_nodes in PHASE_NODES:
        n_blocks = n_nodes // BN
        nchunks = n_nodes * DEG // CHUNK
        feats = _compute_feats(node_features, coordT, Wp, bp, sel,
                               n_blocks, blk_off)
        feats3s.append(feats.reshape(nchunks, CHUNK, UNITS))
        scs.append(_sc_scatter(chunk0, nchunks))
        blk_off += n_blocks
        chunk0 += nchunks
    partials = [sc(f3, idx2) for sc, f3 in zip(scs, feats3s)]
    return _combine(partials)


# speedup vs baseline: 1.0917x; 1.0917x over previous
import functools

import jax
import jax.numpy as jnp
from jax import lax
from jax.experimental import pallas as pl
from jax.experimental.pallas import tpu as pltpu
from jax.experimental.pallas import tpu_sc as plsc

N_NODES = 10000
DEG = 32
E = N_NODES * DEG
D_FEAT = 128
D_COORD = 4
UNITS = 128

NC = 2
NS = 16
NW = NC * NS
CHUNK = 128
NCHUNK_TOTAL = E // CHUNK
N_PAD = 10240
ROWS_PER_TILE = N_PAD // NS
IDX_WIN = 88
NCHUNK_PAD = 2552

BN = 400
BE = BN * DEG

PHASE_NODES = (5200, 4800)


def _feats_body(x_ref, wp_ref, bp_ref, coord_ref, sel_ref, out_ref):
    x = x_ref[...]
    hp = jnp.dot(x, wp_ref[...], preferred_element_type=jnp.float32)
    hp = hp + bp_ref[...]
    cdall = jax.lax.dot_general(
        coord_ref[...], sel_ref[...], (((0,), (0,)), ((), ())),
        preferred_element_type=jnp.float32)
    acc = jnp.zeros((BE, UNITS), jnp.float32)
    for d in range(D_COORD):
        hd = hp[:, d * UNITS:(d + 1) * UNITS]
        hrep = jnp.broadcast_to(hd[:, None, :], (BN, DEG, UNITS))
        hrep = hrep.reshape(BE, UNITS)
        acc = acc + hrep * cdall[:, d * UNITS:(d + 1) * UNITS]
    out_ref[...] = acc


def _compute_feats(node_features, coordT, Wp, bp, sel, n_blocks, blk_off):
    return pl.pallas_call(
        _feats_body,
        grid=(n_blocks,),
        in_specs=[
            pl.BlockSpec((BN, D_FEAT), lambda i: (i + blk_off, 0)),
            pl.BlockSpec((D_FEAT, D_COORD * UNITS), lambda i: (0, 0)),
            pl.BlockSpec((1, D_COORD * UNITS), lambda i: (0, 0)),
            pl.BlockSpec((D_COORD, BE), lambda i: (0, i + blk_off)),
            pl.BlockSpec((D_COORD, D_COORD * UNITS), lambda i: (0, 0)),
        ],
        out_specs=pl.BlockSpec((BE, UNITS), lambda i: (i, 0)),
        out_shape=jax.ShapeDtypeStruct((n_blocks * BE, UNITS), jnp.float32),
    )(node_features, Wp, bp, coordT, sel)


def _sc_scatter_body(chunk0, nchunk, nextra,
                     feats_hbm, idx_hbm, out_hbm,
                     idx_v, buf_v, acc_sh, sem0, sem1):
    cid = lax.axis_index("c")
    sid = lax.axis_index("s")
    wid = cid * NS + sid

    zv = jnp.zeros((16,), jnp.float32)

    @pl.loop(0, 128)
    def _zero_rows(r):
        for s in range(2):
            for c in range(UNITS // 16):
                buf_v[s, r, pl.ds(c * 16, 16)] = zv

    row0 = pl.multiple_of(sid * ROWS_PER_TILE, 8)
    for k in range(ROWS_PER_TILE // 128):
        pltpu.sync_copy(buf_v.at[k % 2],
                        acc_sh.at[pl.ds(row0 + k * 128, 128)])

    base = nchunk * wid + jnp.minimum(wid, nextra)

    gbase = base + chunk0
    base_al = pl.multiple_of((gbase // 8) * 8, 8)
    dlt = gbase - base_al
    pltpu.sync_copy(idx_hbm.at[pl.ds(base_al, IDX_WIN)], idx_v)

    plsc.subcore_barrier()

    neven = nchunk & ~1
    b0 = buf_v.at[0]
    b1 = buf_v.at[1]
    pltpu.async_copy(feats_hbm.at[base], b0, sem0)
    pltpu.async_copy(feats_hbm.at[base + 1], b1, sem1)

    @pl.loop(0, neven // 2 - 1)
    def _main(j):
        c = j * 2
        pltpu.make_async_copy(feats_hbm.at[base + c], b0, sem0).wait()
        pltpu.sync_copy(b0, acc_sh.at[idx_v.at[dlt + c]], add=True)
        pltpu.async_copy(feats_hbm.at[base + c + 2], b0, sem0)
        pltpu.make_async_copy(feats_hbm.at[base + c + 1], b1, sem1).wait()
        pltpu.sync_copy(b1, acc_sh.at[idx_v.at[dlt + c + 1]], add=True)
        pltpu.async_copy(feats_hbm.at[base + c + 3], b1, sem1)

    last = neven - 2
    pltpu.make_async_copy(feats_hbm.at[base + last], b0, sem0).wait()
    pltpu.sync_copy(b0, acc_sh.at[idx_v.at[dlt + last]], add=True)
    if nchunk & 1:
        pltpu.async_copy(feats_hbm.at[base + neven], b0, sem0)
    pltpu.make_async_copy(feats_hbm.at[base + last + 1], b1, sem1).wait()
    pltpu.sync_copy(b1, acc_sh.at[idx_v.at[dlt + last + 1]], add=True)
    if nchunk & 1:
        pltpu.make_async_copy(feats_hbm.at[base + neven], b0, sem0).wait()
        pltpu.sync_copy(b0, acc_sh.at[idx_v.at[dlt + neven]], add=True)

    if nextra:
        @pl.when(wid < nextra)
        def _extra():
            pltpu.sync_copy(feats_hbm.at[base + nchunk], b0)
            pltpu.sync_copy(b0, acc_sh.at[idx_v.at[dlt + nchunk]], add=True)

    plsc.subcore_barrier()

    pltpu.sync_copy(acc_sh.at[pl.ds(row0, ROWS_PER_TILE)],
                    out_hbm.at[cid, pl.ds(row0, ROWS_PER_TILE)])


@functools.lru_cache(maxsize=4)
def _sc_scatter(chunk0, nchunks_phase):
    nchunk = nchunks_phase // NW
    nextra = nchunks_phase - nchunk * NW
    return pl.kernel(
        functools.partial(_sc_scatter_body, chunk0, nchunk, nextra),
        out_type=jax.ShapeDtypeStruct((NC, N_PAD, UNITS), jnp.float32),
        mesh=plsc.VectorSubcoreMesh(
            core_axis_name="c", subcore_axis_name="s",
            num_cores=NC, num_subcores=NS),
        scratch_types=[
            pltpu.VMEM((IDX_WIN, CHUNK), jnp.int32),
            pltpu.VMEM((2, CHUNK, UNITS), jnp.float32),
            pltpu.VMEM_SHARED((N_PAD, UNITS), jnp.float32),
            pltpu.SemaphoreType.DMA,
            pltpu.SemaphoreType.DMA,
        ],
    )


def _combine_body(*refs):
    o_ref = refs[-1]
    acc = None
    for p_ref in refs[:-1]:
        t = p_ref[0] + p_ref[1]
        acc = t if acc is None else acc + t
    o_ref[...] = acc


def _combine(partials):
    nb = 10
    rb = N_NODES // nb
    return pl.pallas_call(
        _combine_body,
        grid=(nb,),
        in_specs=[pl.BlockSpec((NC, rb, UNITS), lambda i: (0, i, 0))
                  for _ in partials],
        out_specs=pl.BlockSpec((rb, UNITS), lambda i: (i, 0)),
        out_shape=jax.ShapeDtypeStruct((N_NODES, UNITS), jnp.float32),
    )(*partials)


def kernel(node_features, coord_features, indices, row_splits, W, b):
    del row_splits
    Wp = W.reshape(D_FEAT, UNITS, D_COORD).transpose(0, 2, 1)
    Wp = Wp.reshape(D_FEAT, D_COORD * UNITS)
    bp = b.reshape(UNITS, D_COORD).T.reshape(1, D_COORD * UNITS)
    sel = jnp.repeat(jnp.eye(D_COORD, dtype=jnp.float32), UNITS, axis=1)
    coordT = coord_features.T

    idx2 = jnp.concatenate(
        [indices, jnp.zeros((NCHUNK_PAD * CHUNK - E,), jnp.int32)]
    ).reshape(NCHUNK_PAD, CHUNK)

    feats3s, scs = [], []
    blk_off = 0
    chunk0 = 0
    for n_nodes in PHASE_NODES:
        n_blocks = n_nodes // BN
        nchunks = n_nodes * DEG // CHUNK
        feats = _compute_feats(node_features, coordT, Wp, bp, sel,
                               n_blocks, blk_off)
        feats3s.append(feats.reshape(nchunks, CHUNK, UNITS))
        scs.append(_sc_scatter(chunk0, nchunks))
        blk_off += n_blocks
        chunk0 += nchunks
    partials = [sc(f3, idx2) for sc, f3 in zip(scs, feats3s)]
    return _combine(partials)
